# Initial kernel scaffold; baseline (speedup 1.0000x reference)
import functools
import jax
import jax.numpy as jnp
from jax import lax
from jax.experimental import pallas as pl
from jax.experimental.pallas import tpu as pltpu
from jax.experimental.pallas import tpu_sc as plsc

N = 100000
E = 1600000
NC, NS, LANES = 2, 16, 16
K = 2000


def _bcast_lane(v, lane):
    idx = jnp.zeros((16,), jnp.int32) + lane
    return lax.gather(
        v, idx[:, None],
        dimension_numbers=lax.GatherDimensionNumbers(
            offset_dims=(), collapsed_slice_dims=(0,), start_index_map=(0,)),
        slice_sizes=(1,), mode=lax.GatherScatterMode.PROMISE_IN_BOUNDS)


def _probe_body(gmax_hbm, src_hbm, dst_hbm, alsrc_hbm, ex_hbm, spart_hbm,
                gmaxv, srcv, dstv, asrcv, exv, zb, s_sh, sem):
    c = lax.axis_index("c")
    s = lax.axis_index("s")
    pltpu.sync_copy(gmax_hbm, gmaxv)
    gv = gmaxv[...]

    zbf = zb.reshape(6400 * 4)

    def zloop(i, _):
        zbf[pl.ds(i * 16, 16)] = jnp.zeros((16,), jnp.float32)
        return 0
    lax.fori_loop(0, 1600, zloop, 0)
    pltpu.sync_copy(zb.at[pl.ds(0, 6250)], s_sh.at[pl.ds(s * 6250, 6250)])
    plsc.subcore_barrier()

    base = c * (E // 2) + s * 50000

    def chunk(k, _):
        off = base + k * K
        pltpu.sync_copy(src_hbm.at[pl.ds(off, K)], srcv)
        pltpu.sync_copy(dst_hbm.at[pl.ds(off, K)], dstv)
        pltpu.async_copy(alsrc_hbm.at[srcv], asrcv, sem).wait()
        af = asrcv.reshape(K * 4)
        ef = exv.reshape(K * 4)

        def jloop(j, _):
            a = af[pl.ds(j * 16, 16)]
            z = a + gv
            m = jnp.where(z > 0, z, 0.2 * z)
            w = _bcast_lane(a, 4 * 0 + c)
            ef[pl.ds(j * 16, 16)] = jnp.exp(a - m) * w
            return 0
        lax.fori_loop(0, K * 4 // 16, jloop, 0)
        pltpu.sync_copy(exv, ex_hbm.at[pl.ds(off, K)])
        pltpu.sync_copy(exv, s_sh.at[dstv], add=True)
        return 0
    lax.fori_loop(0, 25, chunk, 0)
    plsc.subcore_barrier()

    @pl.when(c == 0)
    def _():
        pltpu.sync_copy(s_sh.at[pl.ds(s * 6250, 6250)],
                        spart_hbm.at[0, pl.ds(s * 6250, 6250)])

    @pl.when(c == 1)
    def _():
        pltpu.sync_copy(s_sh.at[pl.ds(s * 6250, 6250)],
                        spart_hbm.at[1, pl.ds(s * 6250, 6250)])


def _sc_probe(gmax, src, dst, alsrc):
    mesh = plsc.VectorSubcoreMesh(core_axis_name="c", subcore_axis_name="s")
    f = pl.kernel(
        _probe_body,
        out_type=[jax.ShapeDtypeStruct((E, 4), jnp.float32),
                  jax.ShapeDtypeStruct((2, N, 4), jnp.float32)],
        mesh=mesh,
        scratch_types=[
            pltpu.VMEM((16,), jnp.float32),
            pltpu.VMEM((K,), jnp.int32),
            pltpu.VMEM((K,), jnp.int32),
            pltpu.VMEM((K, 4), jnp.float32),
            pltpu.VMEM((K, 4), jnp.float32),
            pltpu.VMEM((6400, 4), jnp.float32),
            pltpu.VMEM_SHARED((N, 4), jnp.float32),
            pltpu.SemaphoreType.DMA,
        ])
    return f(gmax, src, dst, alsrc)


def kernel(x, edge_index, batch, Wq, Wk, Wv, W1, a_src1, a_dst1, b1, W2,
           a_src2, a_dst2, b2, W3, a_src3, a_dst3, b3, W4, a_src4, a_dst4,
           b4, Wfc, bfc):
    src = edge_index[0]
    dst = edge_index[1]
    gmax = jnp.zeros((16,), jnp.float32)
    alsrc = jnp.zeros((N, 4), jnp.float32)
    ex, spart = _sc_probe(gmax, src, dst, alsrc)
    logits = jnp.zeros((64, 2), jnp.float32) + ex[0, 0] + spart[0, 0, 0]
    return (logits, jax.nn.softmax(logits, axis=1))


# speedup vs baseline: 60.7705x; 60.7705x over previous
import jax
import jax.numpy as jnp
import numpy as np
from jax import lax
from jax.experimental import pallas as pl
from jax.experimental.pallas import tpu as pltpu
from jax.experimental.pallas import tpu_sc as plsc

N = 100000
NP_ = 100352
RPT = 6272
E = 1600000
K = 2000
K2 = 800
NTC = 32
BN = NP_ // NTC
NTCA = 64
BNA = NP_ // NTCA
EPS = 1e-16

_VLIM = pltpu.CompilerParams(vmem_limit_bytes=100 * 1024 * 1024)
_SCPARAMS = pltpu.CompilerParams(use_tc_tiling_on_sc=False,
                                 needs_layout_passes=False)


def _leaky(x, a):
    return jnp.where(x > 0, x, a * x)


def _dot(a, b):
    return jnp.dot(a, b, precision=lax.Precision.HIGHEST)


def _dot_ref(a, b):
    return jnp.dot(a.astype(jnp.bfloat16), b.astype(jnp.bfloat16),
                   preferred_element_type=jnp.float32)


def _full(*shape):
    return pl.BlockSpec(shape, lambda i: tuple(0 for _ in shape))


def _prep_tail(h, As_ref, Ad_ref, h64_ref, alm_ref, pm_ref):
    alsrc = _dot(h, As_ref[...])
    aldst = _dot(h, Ad_ref[...])
    h64_ref[...] = h
    alm_ref[...] = jnp.concatenate([alsrc, aldst], axis=1)
    pm_ref[...] = jnp.max(alsrc, axis=0).reshape(1, 1, 4)


def _tca_body(x_ref, wq_ref, wk_ref, wv_ref, w1_ref, As_ref, Ad_ref,
              h64_ref, alm_ref, pm_ref):
    xb = x_ref[...]
    ai = xb.reshape(BNA // 32, 32, 10).transpose(0, 2, 1)
    a2 = ai.reshape(BNA // 32 * 10, 32)
    Q = _dot(a2, wq_ref[...]).reshape(BNA // 32, 10, 32)
    Kk = _dot(a2, wk_ref[...]).reshape(BNA // 32, 10, 32)
    V = _dot(a2, wv_ref[...]).reshape(BNA // 32, 10, 32)
    S = lax.dot_general(Q, Kk, (((2,), (2,)), ((0,), (0,))),
                        precision=lax.Precision.HIGHEST)
    S = S - jnp.max(S, axis=2, keepdims=True)
    P = jnp.exp(S)
    P = P / jnp.sum(P, axis=2, keepdims=True)
    out = lax.dot_general(P, V, (((2,), (1,)), ((0,), (0,))),
                          precision=lax.Precision.HIGHEST)
    xg = out.transpose(0, 2, 1).reshape(BNA, 10)
    h = _dot_ref(xg, w1_ref[...])
    _prep_tail(h, As_ref, Ad_ref, h64_ref, alm_ref, pm_ref)


def _tc_attn_prep(xp, Wq, Wk, Wv, W1, As, Ad):
    return pl.pallas_call(
        _tca_body,
        grid=(NTCA,),
        compiler_params=_VLIM,
        in_specs=[
            pl.BlockSpec((BNA, 10), lambda i: (i, 0)),
            _full(32, 32), _full(32, 32), _full(32, 32),
            _full(10, 64), _full(64, 4), _full(64, 4),
        ],
        out_specs=[
            pl.BlockSpec((BNA, 64), lambda i: (i, 0)),
            pl.BlockSpec((BNA, 8), lambda i: (i, 0)),
            pl.BlockSpec((1, 1, 4), lambda i: (i, 0, 0)),
        ],
        out_shape=[
            jax.ShapeDtypeStruct((NP_, 64), jnp.float32),
            jax.ShapeDtypeStruct((NP_, 8), jnp.float32),
            jax.ShapeDtypeStruct((NTCA, 1, 4), jnp.float32),
        ],
    )(xp, Wq, Wk, Wv, W1, As, Ad)


def _finalize_block(acc_ref, s0_ref, s1_ref, alm_ref, gm_ref,
                    h64_ref, b_ref):
    acc = acc_ref[...]
    s0 = s0_ref[...][:, 0:4]
    s1 = s1_ref[...][:, 0:4]
    alm = alm_ref[...]
    asr = alm[:, 0:4]
    ads = alm[:, 4:8]
    M = _leaky(gm_ref[...] + ads, 0.2)
    exself = jnp.exp(_leaky(asr + ads, 0.2) - M)
    stot = s0 + s1 + exself
    inv = 1.0 / (stot + EPS)
    exw = jnp.broadcast_to(exself[:, :, None], (BN, 4, 16)).reshape(BN, 64)
    invw = jnp.broadcast_to(inv[:, :, None], (BN, 4, 16)).reshape(BN, 64)
    out = (acc + exw * h64_ref[...]) * invw + b_ref[...]
    return _leaky(out, 0.01)


def _tcb_body(acc_ref, s0_ref, s1_ref, alm_ref, gm_ref, h64_ref,
              b_ref, w_ref, As_ref, Ad_ref,
              h64o_ref, almo_ref, pm_ref):
    hout = _finalize_block(acc_ref, s0_ref, s1_ref, alm_ref, gm_ref,
                           h64_ref, b_ref)
    h = _dot_ref(hout, w_ref[...])
    _prep_tail(h, As_ref, Ad_ref, h64o_ref, almo_ref, pm_ref)


def _tc_fin_prep(acc64, s0, s1, alm, gmax, h64, bvec, Wn, Asn, Adn):
    return pl.pallas_call(
        _tcb_body,
        grid=(NTC,),
        compiler_params=_VLIM,
        in_specs=[
            pl.BlockSpec((BN, 64), lambda i: (i, 0)),
            pl.BlockSpec((BN, 8), lambda i: (i, 0)),
            pl.BlockSpec((BN, 8), lambda i: (i, 0)),
            pl.BlockSpec((BN, 8), lambda i: (i, 0)),
            _full(1, 4),
            pl.BlockSpec((BN, 64), lambda i: (i, 0)),
            _full(1, 64),
            _full(64, 64), _full(64, 4), _full(64, 4),
        ],
        out_specs=[
            pl.BlockSpec((BN, 64), lambda i: (i, 0)),
            pl.BlockSpec((BN, 8), lambda i: (i, 0)),
            pl.BlockSpec((1, 1, 4), lambda i: (i, 0, 0)),
        ],
        out_shape=[
            jax.ShapeDtypeStruct((NP_, 64), jnp.float32),
            jax.ShapeDtypeStruct((NP_, 8), jnp.float32),
            jax.ShapeDtypeStruct((NTC, 1, 4), jnp.float32),
        ],
    )(acc64, s0, s1, alm, gmax, h64, bvec, Wn, Asn, Adn)


def _tcc_body(acc_ref, s0_ref, s1_ref, alm_ref, gm_ref, h64_ref,
              b_ref, bat_ref, wfc_ref, bfc_ref, lg_ref, pr_ref, accp, cntp):
    i = pl.program_id(0)
    hout = _finalize_block(acc_ref, s0_ref, s1_ref, alm_ref, gm_ref,
                           h64_ref, b_ref)
    bv = bat_ref[0, 0, :]
    oh = (bv[:, None] == lax.broadcasted_iota(jnp.int32, (BN, 64), 1))
    oh = oh.astype(jnp.float32)
    pc = lax.dot_general(oh, hout, (((0,), (0,)), ((), ())),
                         precision=lax.Precision.HIGHEST)
    cn = jnp.sum(oh, axis=0, keepdims=True)

    @pl.when(i == 0)
    def _():
        accp[...] = pc
        cntp[...] = cn

    @pl.when(i != 0)
    def _():
        accp[...] = accp[...] + pc
        cntp[...] = cntp[...] + cn

    @pl.when(i == NTC - 1)
    def _():
        cnt = jnp.maximum(cntp[...], 1.0).reshape(64, 1)
        pooled = accp[...] / cnt
        lg = _dot(pooled, wfc_ref[...]) + bfc_ref[...]
        m = jnp.max(lg, axis=1, keepdims=True)
        p = jnp.exp(lg - m)
        p = p / jnp.sum(p, axis=1, keepdims=True)
        lg_ref[...] = lg
        pr_ref[...] = p


def _tc_fin_pool(acc64, s0, s1, alm, gmax, h64, bvec, batch3,
                 Wfc, bfc):
    return pl.pallas_call(
        _tcc_body,
        grid=(NTC,),
        compiler_params=_VLIM,
        in_specs=[
            pl.BlockSpec((BN, 64), lambda i: (i, 0)),
            pl.BlockSpec((BN, 8), lambda i: (i, 0)),
            pl.BlockSpec((BN, 8), lambda i: (i, 0)),
            pl.BlockSpec((BN, 8), lambda i: (i, 0)),
            _full(1, 4),
            pl.BlockSpec((BN, 64), lambda i: (i, 0)),
            _full(1, 64),
            pl.BlockSpec((1, 1, BN), lambda i: (i, 0, 0)),
            _full(64, 2), _full(1, 2),
        ],
        out_specs=[_full(64, 2), _full(64, 2)],
        out_shape=[
            jax.ShapeDtypeStruct((64, 2), jnp.float32),
            jax.ShapeDtypeStruct((64, 2), jnp.float32),
        ],
        scratch_shapes=[
            pltpu.VMEM((64, 64), jnp.float32),
            pltpu.VMEM((1, 64), jnp.float32),
        ],
    )(acc64, s0, s1, alm, gmax, h64, bvec, batch3, Wfc, bfc)


def _lane_idx():
    i = lax.iota(jnp.int32, 16)
    return lax.shift_right_logical(i, 2), lax.bitwise_and(i, 3)


def _sc1_body(gmax_hbm, zeros8_hbm, src_hbm, dst_hbm, alm_hbm,
              ex_hbm, sp_hbm,
              gmaxv, srcv, dstv, msrcv, mdstv, exv4, exv8, s_sh, sem):
    c = lax.axis_index("c")
    s = lax.axis_index("s")
    pltpu.sync_copy(gmax_hbm, gmaxv)
    pltpu.sync_copy(zeros8_hbm, s_sh.at[pl.ds(s * RPT, RPT)])
    r4, c4 = _lane_idx()
    c4p4 = c4 + 4

    def z8loop(j, _):
        plsc.store_scatter(exv8, [j * 4 + r4, c4p4],
                           jnp.zeros((16,), jnp.float32))
        return 0
    lax.fori_loop(0, K // 4, z8loop, 0)
    plsc.subcore_barrier()

    base = c * (E // 2) + s * (E // 32)

    def chunk(k, _):
        off = base + k * K
        pltpu.sync_copy(src_hbm.at[pl.ds(off, K)], srcv)
        pltpu.sync_copy(dst_hbm.at[pl.ds(off, K)], dstv)
        pltpu.async_copy(alm_hbm.at[srcv], msrcv, sem).wait()
        pltpu.async_copy(alm_hbm.at[dstv], mdstv, sem).wait()
        gv = gmaxv[...]

        def jloop(j, _):
            row = j * 4 + r4
            a = plsc.load_gather(msrcv, [row, c4])
            b = plsc.load_gather(mdstv, [row, c4p4])
            z = a + b
            e = jnp.where(z > 0, z, 0.2 * z)
            z2 = gv + b
            m = jnp.where(z2 > 0, z2, 0.2 * z2)
            ex = jnp.exp(e - m)
            plsc.store_scatter(exv4, [row, c4], ex)
            plsc.store_scatter(exv8, [row, c4], ex)
            return 0
        lax.fori_loop(0, K // 4, jloop, 0)
        pltpu.sync_copy(exv4, ex_hbm.at[pl.ds(off, K)])
        pltpu.sync_copy(exv8, s_sh.at[dstv], add=True)
        return 0
    lax.fori_loop(0, E // 32 // K, chunk, 0)
    plsc.subcore_barrier()

    @pl.when(c == 0)
    def _():
        pltpu.sync_copy(s_sh.at[pl.ds(s * RPT, RPT)],
                        sp_hbm.at[0, pl.ds(s * RPT, RPT)])

    @pl.when(c == 1)
    def _():
        pltpu.sync_copy(s_sh.at[pl.ds(s * RPT, RPT)],
                        sp_hbm.at[1, pl.ds(s * RPT, RPT)])


def _sc_pass1(gmax16, zeros8, src, dst, alm):
    mesh = plsc.VectorSubcoreMesh(core_axis_name="c", subcore_axis_name="s")
    f = pl.kernel(
        _sc1_body,
        out_type=[jax.ShapeDtypeStruct((E, 4), jnp.float32),
                  jax.ShapeDtypeStruct((2, NP_, 8), jnp.float32)],
        mesh=mesh,
        compiler_params=_SCPARAMS,
        scratch_types=[
            pltpu.VMEM((16,), jnp.float32),
            pltpu.VMEM((K,), jnp.int32),
            pltpu.VMEM((K,), jnp.int32),
            pltpu.VMEM((K, 8), jnp.float32),
            pltpu.VMEM((K, 8), jnp.float32),
            pltpu.VMEM((K, 4), jnp.float32),
            pltpu.VMEM((K, 8), jnp.float32),
            pltpu.VMEM_SHARED((NP_, 8), jnp.float32),
            pltpu.SemaphoreType.DMA,
        ])
    return f(gmax16, zeros8, src, dst, alm)


def _sc2_body(zeros16_hbm, src_hbm, dst_hbm, ex_hbm, h4_hbm,
              acc_hbm,
              srcv, sadjv, dstv, exv, rows, acc_sh, sem):
    c = lax.axis_index("c")
    s = lax.axis_index("s")

    for gi in range(2):
        pltpu.sync_copy(zeros16_hbm, acc_sh.at[pl.ds(s * RPT, RPT)])
        plsc.subcore_barrier()

        def chunk(k, _):
            g = c * 2 + gi
            off = s * (E // 16) + k * K2
            pltpu.sync_copy(src_hbm.at[pl.ds(off, K2)], srcv)
            pltpu.sync_copy(dst_hbm.at[pl.ds(off, K2)], dstv)
            pltpu.sync_copy(ex_hbm.at[pl.ds(off, K2)], exv)

            def iloop(i, _):
                sl = pl.ds(i * 16, 16)
                sadjv[sl] = srcv[sl] * 4 + g
                return 0
            lax.fori_loop(0, K2 // 16, iloop, 0)
            pltpu.async_copy(h4_hbm.at[sadjv], rows, sem).wait()
            zi = jnp.zeros((16,), jnp.int32)
            gvec = zi + g

            def eloop(e, _):
                w = plsc.load_gather(exv, [zi + e, gvec])
                rows[e, :] = rows[e, :] * w
                return 0
            lax.fori_loop(0, K2, eloop, 0)
            pltpu.sync_copy(rows, acc_sh.at[dstv], add=True)
            return 0
        lax.fori_loop(0, E // 16 // K2, chunk, 0)
        plsc.subcore_barrier()

        @pl.when(c == 0)
        def _():
            pltpu.sync_copy(acc_sh.at[pl.ds(s * RPT, RPT)],
                            acc_hbm.at[pl.ds(s * RPT, RPT), gi])

        @pl.when(c == 1)
        def _():
            pltpu.sync_copy(acc_sh.at[pl.ds(s * RPT, RPT)],
                            acc_hbm.at[pl.ds(s * RPT, RPT), 2 + gi])


def _sc_pass2(zeros16, src, dst, ex, h4):
    mesh = plsc.VectorSubcoreMesh(core_axis_name="c", subcore_axis_name="s")
    f = pl.kernel(
        _sc2_body,
        out_type=[jax.ShapeDtypeStruct((NP_, 4, 16), jnp.float32)],
        mesh=mesh,
        compiler_params=_SCPARAMS,
        scratch_types=[
            pltpu.VMEM((K2,), jnp.int32),
            pltpu.VMEM((K2,), jnp.int32),
            pltpu.VMEM((K2,), jnp.int32),
            pltpu.VMEM((K2, 4), jnp.float32),
            pltpu.VMEM((K2, 16), jnp.float32),
            pltpu.VMEM_SHARED((NP_, 16), jnp.float32),
            pltpu.SemaphoreType.DMA,
        ])
    return f(zeros16, src, dst, ex, h4)[0]


_KRON = np.kron(np.eye(4), np.ones((16, 1))).astype(np.float32)


def kernel(x, edge_index, batch, Wq, Wk, Wv, W1, a_src1, a_dst1, b1, W2,
           a_src2, a_dst2, b2, W3, a_src3, a_dst3, b3, W4, a_src4, a_dst4,
           b4, Wfc, bfc):
    xp = jnp.pad(x, ((0, NP_ - N), (0, 0)))
    batch3 = jnp.pad(batch, (0, NP_ - N),
                     constant_values=64).reshape(NTC, 1, BN)
    src = edge_index[0]
    dst = edge_index[1]
    zeros8 = jnp.zeros((RPT, 8), jnp.float32)
    zeros16 = jnp.zeros((RPT, 16), jnp.float32)

    def head_mats(a_src, a_dst, H):
        if H == 4:
            As = a_src.reshape(64, 1) * _KRON
            Ad = a_dst.reshape(64, 1) * _KRON
        else:
            As = jnp.tile(a_src.reshape(64, 1), (1, 4))
            Ad = jnp.tile(a_dst.reshape(64, 1), (1, 4))
        return As, Ad

    As1, Ad1 = head_mats(a_src1, a_dst1, 4)
    As2, Ad2 = head_mats(a_src2, a_dst2, 4)
    As3, Ad3 = head_mats(a_src3, a_dst3, 4)
    As4, Ad4 = head_mats(a_src4, a_dst4, 1)

    h64, alm, pmax = _tc_attn_prep(xp, Wq, Wk, Wv, W1, As1, Ad1)

    layer_tail = [(W2, As2, Ad2, b1), (W3, As3, Ad3, b2), (W4, As4, Ad4, b3)]
    logits = probas = None
    for li in range(4):
        gmax = jnp.max(pmax, axis=0).reshape(1, 4)
        gmax16 = jnp.tile(gmax.reshape(4), 4)
        ex, spb = _sc_pass1(gmax16, zeros8, src, dst, alm)
        sp0 = spb[0]
        sp1 = spb[1]
        accb = _sc_pass2(zeros16, src, dst, ex, h64.reshape(4 * NP_, 16))
        acc64 = accb.reshape(NP_, 64)
        if li < 3:
            Wn, Asn, Adn, bl = layer_tail[li]
            h64, alm, pmax = _tc_fin_prep(
                acc64, sp0, sp1, alm, gmax, h64, bl.reshape(1, 64),
                Wn, Asn, Adn)
        else:
            logits, probas = _tc_fin_pool(
                acc64, sp0, sp1, alm, gmax, h64, b4.reshape(1, 64),
                batch3, Wfc, bfc.reshape(1, 2))
    return (logits, probas)
